# Initial kernel scaffold; baseline (speedup 1.0000x reference)
import jax
import jax.numpy as jnp
from jax.experimental import pallas as pl


def kernel(x, Wr, br, We, be):
    raise NotImplementedError("write your pallas kernel here")


# speedup vs baseline: 4.1266x; 4.1266x over previous
import functools

import jax
import jax.numpy as jnp
from jax import lax
from jax.experimental import pallas as pl
from jax.experimental.pallas import tpu as pltpu
from jax.experimental.pallas import tpu_sc as plsc

N, D, E = 2048, 768, 64
TB = 128
BLK = 128
NC, NS = 2, 16
NW = NC * NS
CH = N // NW


def _router_body(x_ref, wr_ref, br_ref, pos_ref, se_ref, sel_s, rank_s):
    nb = N // TB
    iota_e = lax.broadcasted_iota(jnp.int32, (TB, E), 1)

    def block1(t, base):
        xb = x_ref[pl.ds(t * TB, TB), :]
        logits = jnp.dot(xb, wr_ref[...], preferred_element_type=jnp.float32)
        logits = logits + br_ref[...]
        m = jnp.max(logits, axis=1, keepdims=True)
        sel = jnp.min(jnp.where(logits >= m, iota_e, E), axis=1, keepdims=True)
        onehot = (iota_e == sel).astype(jnp.float32)
        r = lax.broadcasted_iota(jnp.int32, (TB, TB), 0)
        c = lax.broadcasted_iota(jnp.int32, (TB, TB), 1)
        tril = (r >= c).astype(jnp.float32)
        cum = jnp.dot(tril, onehot, preferred_element_type=jnp.float32) + base
        rank = jnp.sum(onehot * cum, axis=1, keepdims=True)
        sel_s[pl.ds(t * TB, TB), :] = sel
        rank_s[pl.ds(t * TB, TB), :] = rank
        return cum[TB - 1:TB, :]

    counts = lax.fori_loop(0, nb, block1, jnp.zeros((1, E), jnp.float32))

    r64 = lax.broadcasted_iota(jnp.int32, (E, E), 0)
    c64 = lax.broadcasted_iota(jnp.int32, (E, E), 1)
    excl = jnp.dot(counts, (r64 < c64).astype(jnp.float32),
                   preferred_element_type=jnp.float32)
    se_ref[0:1, :] = excl.astype(jnp.int32)
    se_ref[1:2, :] = (excl + counts).astype(jnp.int32)

    def block2(t, _):
        sel = sel_s[pl.ds(t * TB, TB), :]
        onehot = (iota_e == sel).astype(jnp.float32)
        offg = jnp.sum(onehot * excl, axis=1, keepdims=True)
        rank = rank_s[pl.ds(t * TB, TB), :]
        pos_ref[pl.ds(t * TB, TB), :] = (offg + rank - 1.0).astype(jnp.int32)
        return 0

    lax.fori_loop(0, nb, block2, 0)


_router = pl.pallas_call(
    _router_body,
    out_shape=(jax.ShapeDtypeStruct((N, 1), jnp.int32),
               jax.ShapeDtypeStruct((2, E), jnp.int32)),
    scratch_shapes=[pltpu.VMEM((N, 1), jnp.int32),
                    pltpu.VMEM((N, 1), jnp.float32)],
)


def _moe_body(starts_ref, ends_ref, x_ref, w_ref, b_ref, out_ref):
    e = pl.program_id(0)

    @pl.when(e == 0)
    def _():
        out_ref[...] = jnp.zeros_like(out_ref)

    start = starts_ref[e]
    end = ends_ref[e]
    w = w_ref[0]
    b = b_ref[0]

    def chunk(c, _):
        rb = c * BLK
        xa = x_ref[pl.ds(rb, BLK), :]
        res = jnp.dot(xa, w, preferred_element_type=jnp.float32) + b
        rows = rb + lax.broadcasted_iota(jnp.int32, (BLK, 1), 0)
        valid = (rows >= start) & (rows < end)
        out_ref[pl.ds(rb, BLK), :] = (
            out_ref[pl.ds(rb, BLK), :] + jnp.where(valid, res, 0.0))
        return 0

    lax.fori_loop(start // BLK, (end - 1) // BLK + 1, chunk, 0)


_moe = pl.pallas_call(
    _moe_body,
    grid_spec=pltpu.PrefetchScalarGridSpec(
        num_scalar_prefetch=2,
        grid=(E,),
        in_specs=[
            pl.BlockSpec((N, D), lambda e, s, t: (0, 0)),
            pl.BlockSpec((1, D, D), lambda e, s, t: (e, 0, 0)),
            pl.BlockSpec((1, 1, D), lambda e, s, t: (e, 0, 0)),
        ],
        out_specs=pl.BlockSpec((N, D), lambda e, s, t: (0, 0)),
    ),
    out_shape=jax.ShapeDtypeStruct((N, D), jnp.float32),
)


@functools.cache
def _make_permute(mode):
    @functools.partial(
        pl.kernel,
        mesh=plsc.VectorSubcoreMesh(core_axis_name="c", subcore_axis_name="s"),
        out_type=jax.ShapeDtypeStruct((N, D), jnp.float32),
        scratch_types=[
            pltpu.VMEM((CH,), jnp.int32),
            pltpu.VMEM((CH, D), jnp.float32),
            pltpu.SemaphoreType.DMA,
        ],
    )
    def k(src, idx, out, idx_v, rows_v, sem):
        wid = lax.axis_index("s") * NC + lax.axis_index("c")
        base = wid * CH
        pltpu.sync_copy(idx.at[pl.ds(base, CH)], idx_v)
        if mode == "scatter":
            pltpu.sync_copy(src.at[pl.ds(base, CH)], rows_v)
            pltpu.async_copy(rows_v, out.at[idx_v], sem).wait()
        else:
            pltpu.async_copy(src.at[idx_v], rows_v, sem).wait()
            pltpu.sync_copy(rows_v, out.at[pl.ds(base, CH)])

    return k


def kernel(x, Wr, br, We, be):
    pos2d, se = _router(x, Wr, br.reshape(1, E))
    pos = pos2d.reshape(N)
    x_sorted = _make_permute("scatter")(x, pos)
    out_sorted = _moe(se[0], se[1], x_sorted, We, be.reshape(E, 1, D))
    return _make_permute("gather")(out_sorted, pos)
